# baseline (speedup 1.0000x reference)
import functools

import jax
import jax.numpy as jnp
from jax import lax
from jax.experimental import pallas as pl
from jax.experimental.pallas import tpu as pltpu
from jax.experimental.pallas import tpu_sc as plsc

N, D, E, K, F = 2048, 1024, 8, 2, 2048
TEMP = 1.0

BT = 256
BLK = 128
G = 40
RPAD = G * BLK


def _router_body(x_ref, wg_ref, probs_ref, topv_ref, topi_ref, wn_ref,
                 ent_ref, mass_ref, ranks_ref, counts_ref, rowstart_ref,
                 nbk_ref, gid_ref, cnt):
    i = pl.program_id(0)

    @pl.when(i == 0)
    def _():
        cnt[...] = jnp.zeros_like(cnt)

    l = jnp.dot(x_ref[...], wg_ref[...], preferred_element_type=jnp.float32)
    l = l / TEMP
    m = jnp.max(l, axis=1, keepdims=True)
    e = jnp.exp(l - m)
    s = jnp.sum(e, axis=1, keepdims=True)
    p = e / s
    probs_ref[...] = p
    iota = lax.broadcasted_iota(jnp.int32, p.shape, 1)
    v1 = jnp.max(p, axis=1, keepdims=True)
    i1 = jnp.min(jnp.where(p == v1, iota, E), axis=1, keepdims=True)
    p2 = jnp.where(iota == i1, -1.0, p)
    v2 = jnp.max(p2, axis=1, keepdims=True)
    i2 = jnp.min(jnp.where(p2 == v2, iota, E), axis=1, keepdims=True)
    topv_ref[...] = jnp.concatenate([v1, v2], axis=1)
    topi_ref[...] = jnp.concatenate([i1, i2], axis=1)
    ws = v1 + v2
    wn_ref[...] = jnp.concatenate([v1 / ws, v2 / ws], axis=1)
    ent_ref[...] = -jnp.sum(p * jnp.log(p + 1e-9), axis=1, keepdims=True)
    mass_ref[...] = ws
    oh = (i1 == iota).astype(jnp.int32) + (i2 == iota).astype(jnp.int32)
    tri = (lax.broadcasted_iota(jnp.int32, (BT, BT), 0)
           > lax.broadcasted_iota(jnp.int32, (BT, BT), 1)).astype(jnp.float32)
    excl = jnp.dot(tri, oh.astype(jnp.float32),
                   preferred_element_type=jnp.float32,
                   precision=lax.Precision.HIGHEST).astype(jnp.int32) + cnt[...]
    r1 = jnp.sum(jnp.where(i1 == iota, excl, 0), axis=1, keepdims=True)
    r2 = jnp.sum(jnp.where(i2 == iota, excl, 0), axis=1, keepdims=True)
    ranks_ref[...] = jnp.concatenate([r1, r2], axis=1)
    cnt[...] = cnt[...] + jnp.sum(oh, axis=0, keepdims=True)
    counts_ref[...] = cnt[...]

    @pl.when(i == N // BT - 1)
    def _():
        nbf = jnp.floor((cnt[...].astype(jnp.float32) + (BLK - 1)) / BLK)
        ut = (lax.broadcasted_iota(jnp.int32, (E, E), 0)
              <= lax.broadcasted_iota(jnp.int32, (E, E), 1)).astype(jnp.float32)
        cum = jnp.dot(nbf, ut, preferred_element_type=jnp.float32,
                      precision=lax.Precision.HIGHEST)
        rowstart_ref[...] = (BLK * (cum - nbf)).astype(jnp.int32)
        nbk_ref[...] = cum.astype(jnp.int32)
        ident = (lax.broadcasted_iota(jnp.int32, (E, E), 0)
                 == lax.broadcasted_iota(jnp.int32, (E, E), 1)).astype(jnp.float32)
        cum_t = lax.dot_general(ident, cum, (((1,), (1,)), ((), ())),
                                preferred_element_type=jnp.float32,
                                precision=lax.Precision.HIGHEST)
        gg = lax.broadcasted_iota(jnp.int32, (E, 64), 1).astype(jnp.float32)
        cmp = (cum_t <= gg).astype(jnp.float32)
        gidf = jnp.dot(jnp.ones((1, E), jnp.float32), cmp,
                       preferred_element_type=jnp.float32,
                       precision=lax.Precision.HIGHEST)
        gid_ref[...] = jnp.minimum(gidf.astype(jnp.int32), E - 1)


def _router(x, Wg):
    return pl.pallas_call(
        _router_body,
        grid=(N // BT,),
        in_specs=[
            pl.BlockSpec((BT, D), lambda i: (i, 0)),
            pl.BlockSpec((D, E), lambda i: (0, 0)),
        ],
        out_specs=[
            pl.BlockSpec((BT, E), lambda i: (i, 0)),
            pl.BlockSpec((BT, K), lambda i: (i, 0)),
            pl.BlockSpec((BT, K), lambda i: (i, 0)),
            pl.BlockSpec((BT, K), lambda i: (i, 0)),
            pl.BlockSpec((BT, 1), lambda i: (i, 0)),
            pl.BlockSpec((BT, 1), lambda i: (i, 0)),
            pl.BlockSpec((BT, K), lambda i: (i, 0)),
            pl.BlockSpec((1, E), lambda i: (0, 0)),
            pl.BlockSpec((1, E), lambda i: (0, 0)),
            pl.BlockSpec((1, E), lambda i: (0, 0)),
            pl.BlockSpec((1, 64), lambda i: (0, 0)),
        ],
        out_shape=[
            jax.ShapeDtypeStruct((N, E), jnp.float32),
            jax.ShapeDtypeStruct((N, K), jnp.float32),
            jax.ShapeDtypeStruct((N, K), jnp.int32),
            jax.ShapeDtypeStruct((N, K), jnp.float32),
            jax.ShapeDtypeStruct((N, 1), jnp.float32),
            jax.ShapeDtypeStruct((N, 1), jnp.float32),
            jax.ShapeDtypeStruct((N, K), jnp.int32),
            jax.ShapeDtypeStruct((1, E), jnp.int32),
            jax.ShapeDtypeStruct((1, E), jnp.int32),
            jax.ShapeDtypeStruct((1, E), jnp.int32),
            jax.ShapeDtypeStruct((1, 64), jnp.int32),
        ],
        scratch_shapes=[pltpu.VMEM((1, E), jnp.int32)],
    )(x, Wg)


def _ffn_body(gid_ref, nblk_ref, xs_ref, w1_ref, b1_ref, w2_ref, b2_ref,
              w_ref, ys_ref):
    g = pl.program_id(0)

    @pl.when(g < nblk_ref[0])
    def _():
        xb = xs_ref[...].astype(jnp.bfloat16)
        h = jnp.dot(xb, w1_ref[0], preferred_element_type=jnp.float32)
        h = jnp.maximum(h + b1_ref[0], 0.0).astype(jnp.bfloat16)
        y = jnp.dot(h, w2_ref[0], preferred_element_type=jnp.float32)
        ys_ref[...] = (y + b2_ref[0]) * w_ref[...]


def _ffn(xs, W1, b1, W2, b2, w_col, gid, nblk):
    grid_spec = pltpu.PrefetchScalarGridSpec(
        num_scalar_prefetch=2,
        grid=(G,),
        in_specs=[
            pl.BlockSpec((BLK, D), lambda g, gid, nblk: (g, 0)),
            pl.BlockSpec((1, D, F), lambda g, gid, nblk: (gid[g], 0, 0)),
            pl.BlockSpec((1, 1, F), lambda g, gid, nblk: (gid[g], 0, 0)),
            pl.BlockSpec((1, F, D), lambda g, gid, nblk: (gid[g], 0, 0)),
            pl.BlockSpec((1, 1, D), lambda g, gid, nblk: (gid[g], 0, 0)),
            pl.BlockSpec((BLK, 1), lambda g, gid, nblk: (g, 0)),
        ],
        out_specs=pl.BlockSpec((BLK, D), lambda g, gid, nblk: (g, 0)),
    )
    return pl.pallas_call(
        _ffn_body,
        grid_spec=grid_spec,
        out_shape=jax.ShapeDtypeStruct((RPAD, D), jnp.float32),
    )(gid, nblk, xs, W1.astype(jnp.bfloat16), b1.reshape(E, 1, F),
      W2.astype(jnp.bfloat16), b2.reshape(E, 1, D), w_col)


_NC, _NS = 2, 16
_NW = _NC * _NS

_G_RPW = RPAD // _NW
_G_CH = 40
_G_NCH = _G_RPW // _G_CH

_C_TPW = N // _NW
_C_CH = 32

@functools.cache
def _sc_kernels():
    mesh = plsc.VectorSubcoreMesh(
        core_axis_name="c", subcore_axis_name="s", num_cores=_NC)

    @functools.partial(
        pl.kernel,
        mesh=mesh,
        compiler_params=pltpu.CompilerParams(needs_layout_passes=False),
        out_type=[
            jax.ShapeDtypeStruct((RPAD, D), jnp.float32),
            jax.ShapeDtypeStruct((RPAD,), jnp.float32),
        ],
        scratch_types=[
            pltpu.VMEM((N * K,), jnp.int32),
            pltpu.VMEM((N * K,), jnp.float32),
            pltpu.VMEM((_G_RPW,), jnp.int32),
            pltpu.VMEM((_G_RPW,), jnp.float32),
            pltpu.VMEM((_G_CH, D), jnp.float32),
            pltpu.VMEM((_G_CH, D), jnp.float32),
            pltpu.SemaphoreType.DMA,
            pltpu.SemaphoreType.DMA,
            pltpu.SemaphoreType.DMA,
            pltpu.SemaphoreType.DMA,
        ],
    )
    def sc_gather(dst_hbm, wn_hbm, x_hbm, xs_hbm, wpad_hbm,
                  dst_v, wnv, tok_v, w_v, r0, r1, sg0, sg1, sw0, sw1):
        wid = lax.axis_index("s") * _NC + lax.axis_index("c")
        base = wid * _G_RPW
        pltpu.sync_copy(dst_hbm, dst_v)
        pltpu.sync_copy(wn_hbm, wnv)
        zi = jnp.zeros((16,), jnp.int32)
        zf = jnp.zeros((16,), jnp.float32)
        for j in range(_G_RPW // 16):
            tok_v[pl.ds(j * 16, 16)] = zi
            w_v[pl.ds(j * 16, 16)] = zf
        lane = lax.iota(jnp.int32, 16)
        for j in range(N * K // 16):
            dvec = dst_v[pl.ds(j * 16, 16)]
            local = dvec - base
            msk = (local >= 0) & (local < _G_RPW)
            lidx = jnp.clip(local, 0, _G_RPW - 1)
            tvec = lax.shift_right_logical(lane + (j * 16), 1)
            plsc.store_scatter(tok_v, [lidx], tvec, mask=msk)
            plsc.store_scatter(w_v, [lidx], wnv[pl.ds(j * 16, 16)], mask=msk)
        pltpu.sync_copy(w_v, wpad_hbm.at[pl.ds(base, _G_RPW)])
        rows, sg, sw = [r0, r1], [sg0, sg1], [sw0, sw1]
        gcp, wcp = [None] * _G_NCH, [None] * _G_NCH
        gcp[0] = pltpu.async_copy(
            x_hbm.at[tok_v.at[pl.ds(0, _G_CH)]], rows[0], sg[0])
        for t in range(_G_NCH):
            if t >= 1:
                wcp[t - 1].wait()
            if t + 1 < _G_NCH:
                gcp[t + 1] = pltpu.async_copy(
                    x_hbm.at[tok_v.at[pl.ds((t + 1) * _G_CH, _G_CH)]],
                    rows[(t + 1) % 2], sg[(t + 1) % 2])
            gcp[t].wait()
            wcp[t] = pltpu.async_copy(
                rows[t % 2], xs_hbm.at[pl.ds(base + t * _G_CH, _G_CH)],
                sw[t % 2])
        wcp[_G_NCH - 1].wait()

    @functools.partial(
        pl.kernel,
        mesh=mesh,
        out_type=jax.ShapeDtypeStruct((N, D), jnp.float32),
        scratch_types=[
            pltpu.VMEM((_C_CH,), jnp.int32),
            pltpu.VMEM((_C_CH,), jnp.int32),
            pltpu.VMEM((_C_CH, D), jnp.float32),
            pltpu.VMEM((_C_CH, D), jnp.float32),
            pltpu.SemaphoreType.DMA,
            pltpu.SemaphoreType.DMA,
        ],
    )
    def sc_combine(p0_hbm, p1_hbm, ys_hbm, out_hbm,
                   i0_v, i1_v, a_v, b_v, s0, s1):
        wid = lax.axis_index("s") * _NC + lax.axis_index("c")
        for t in range(_C_TPW // _C_CH):
            base = wid * _C_TPW + t * _C_CH
            pltpu.sync_copy(p0_hbm.at[pl.ds(base, _C_CH)], i0_v)
            pltpu.sync_copy(p1_hbm.at[pl.ds(base, _C_CH)], i1_v)
            cp0 = pltpu.async_copy(ys_hbm.at[i0_v], a_v, s0)
            cp1 = pltpu.async_copy(ys_hbm.at[i1_v], b_v, s1)
            cp0.wait()
            cp1.wait()

            def _row(r, carry):
                def _col(c, cc):
                    sl = pl.ds(c * 16, 16)
                    a_v[r, sl] = a_v[r, sl] + b_v[r, sl]
                    return cc
                return lax.fori_loop(0, D // 16, _col, carry, unroll=4)

            lax.fori_loop(0, _C_CH, _row, 0)
            pltpu.sync_copy(a_v, out_hbm.at[pl.ds(base, _C_CH)])

    return sc_gather, sc_combine


def _dispatch(topi, ranks, rowstart, nbk, gid64):
    e_flat = topi.reshape(-1)
    dst = rowstart[0][e_flat] + ranks.reshape(-1)
    gid = gid64[0, :G]
    nblk = nbk[0, E - 1:]
    p = dst.reshape(N, K)
    return dst, gid, nblk, p[:, 0], p[:, 1]


def kernel(x, Wg, W1, b1, W2, b2):
    (probs, topv, topi, wn, ent, mass, ranks, counts,
     rowstart, nbk, gid64) = _router(x, Wg)
    dst, gid, nblk, p0, p1 = _dispatch(topi, ranks, rowstart, nbk, gid64)
    sc_gather, sc_combine = _sc_kernels()
    xs, w_pad = sc_gather(dst, wn.reshape(-1), x)
    ys = _ffn(xs, W1, b1, W2, b2, w_pad.reshape(RPAD, 1), gid, nblk)
    out = sc_combine(p0, p1, ys)
    return (out, probs, topi, topv, ent.reshape(N), mass.reshape(N))


# speedup vs baseline: 1.4504x; 1.4504x over previous
import functools

import jax
import jax.numpy as jnp
from jax import lax
from jax.experimental import pallas as pl
from jax.experimental.pallas import tpu as pltpu
from jax.experimental.pallas import tpu_sc as plsc

N, D, E, K, F = 2048, 1024, 8, 2, 2048
TEMP = 1.0

BT = 256
BLK = 128
G = 40
RPAD = G * BLK


def _router_body(x_ref, wg_ref, probs_ref, topv_ref, topi_ref, wn_ref,
                 ent_ref, mass_ref, ranks_ref, counts_ref, rowstart_ref,
                 nbk_ref, gid_ref, cnt):
    i = pl.program_id(0)

    @pl.when(i == 0)
    def _():
        cnt[...] = jnp.zeros_like(cnt)

    l = jnp.dot(x_ref[...], wg_ref[...], preferred_element_type=jnp.float32)
    l = l / TEMP
    m = jnp.max(l, axis=1, keepdims=True)
    e = jnp.exp(l - m)
    s = jnp.sum(e, axis=1, keepdims=True)
    p = e / s
    probs_ref[...] = p
    iota = lax.broadcasted_iota(jnp.int32, p.shape, 1)
    v1 = jnp.max(p, axis=1, keepdims=True)
    i1 = jnp.min(jnp.where(p == v1, iota, E), axis=1, keepdims=True)
    p2 = jnp.where(iota == i1, -1.0, p)
    v2 = jnp.max(p2, axis=1, keepdims=True)
    i2 = jnp.min(jnp.where(p2 == v2, iota, E), axis=1, keepdims=True)
    topv_ref[...] = jnp.concatenate([v1, v2], axis=1)
    topi_ref[...] = jnp.concatenate([i1, i2], axis=1)
    ws = v1 + v2
    wn_ref[...] = jnp.concatenate([v1 / ws, v2 / ws], axis=1)
    ent_ref[...] = -jnp.sum(p * jnp.log(p + 1e-9), axis=1, keepdims=True)
    mass_ref[...] = ws
    oh = (i1 == iota).astype(jnp.int32) + (i2 == iota).astype(jnp.int32)
    tri = (lax.broadcasted_iota(jnp.int32, (BT, BT), 0)
           > lax.broadcasted_iota(jnp.int32, (BT, BT), 1)).astype(jnp.float32)
    excl = jnp.dot(tri, oh.astype(jnp.float32),
                   preferred_element_type=jnp.float32,
                   precision=lax.Precision.HIGHEST).astype(jnp.int32) + cnt[...]
    r1 = jnp.sum(jnp.where(i1 == iota, excl, 0), axis=1, keepdims=True)
    r2 = jnp.sum(jnp.where(i2 == iota, excl, 0), axis=1, keepdims=True)
    ranks_ref[...] = jnp.concatenate([r1, r2], axis=1)
    cnt[...] = cnt[...] + jnp.sum(oh, axis=0, keepdims=True)
    counts_ref[...] = cnt[...]

    @pl.when(i == N // BT - 1)
    def _():
        nbf = jnp.floor((cnt[...].astype(jnp.float32) + (BLK - 1)) / BLK)
        ut = (lax.broadcasted_iota(jnp.int32, (E, E), 0)
              <= lax.broadcasted_iota(jnp.int32, (E, E), 1)).astype(jnp.float32)
        cum = jnp.dot(nbf, ut, preferred_element_type=jnp.float32,
                      precision=lax.Precision.HIGHEST)
        rowstart_ref[...] = (BLK * (cum - nbf)).astype(jnp.int32)
        nbk_ref[...] = cum.astype(jnp.int32)
        ident = (lax.broadcasted_iota(jnp.int32, (E, E), 0)
                 == lax.broadcasted_iota(jnp.int32, (E, E), 1)).astype(jnp.float32)
        cum_t = lax.dot_general(ident, cum, (((1,), (1,)), ((), ())),
                                preferred_element_type=jnp.float32,
                                precision=lax.Precision.HIGHEST)
        gg = lax.broadcasted_iota(jnp.int32, (E, 64), 1).astype(jnp.float32)
        cmp = (cum_t <= gg).astype(jnp.float32)
        gidf = jnp.dot(jnp.ones((1, E), jnp.float32), cmp,
                       preferred_element_type=jnp.float32,
                       precision=lax.Precision.HIGHEST)
        gid_ref[...] = jnp.minimum(gidf.astype(jnp.int32), E - 1)


def _router(x, Wg):
    return pl.pallas_call(
        _router_body,
        grid=(N // BT,),
        in_specs=[
            pl.BlockSpec((BT, D), lambda i: (i, 0)),
            pl.BlockSpec((D, E), lambda i: (0, 0)),
        ],
        out_specs=[
            pl.BlockSpec((BT, E), lambda i: (i, 0)),
            pl.BlockSpec((BT, K), lambda i: (i, 0)),
            pl.BlockSpec((BT, K), lambda i: (i, 0)),
            pl.BlockSpec((BT, K), lambda i: (i, 0)),
            pl.BlockSpec((BT, 1), lambda i: (i, 0)),
            pl.BlockSpec((BT, 1), lambda i: (i, 0)),
            pl.BlockSpec((BT, K), lambda i: (i, 0)),
            pl.BlockSpec((1, E), lambda i: (0, 0)),
            pl.BlockSpec((1, E), lambda i: (0, 0)),
            pl.BlockSpec((1, E), lambda i: (0, 0)),
            pl.BlockSpec((1, 64), lambda i: (0, 0)),
        ],
        out_shape=[
            jax.ShapeDtypeStruct((N, E), jnp.float32),
            jax.ShapeDtypeStruct((N, K), jnp.float32),
            jax.ShapeDtypeStruct((N, K), jnp.int32),
            jax.ShapeDtypeStruct((N, K), jnp.float32),
            jax.ShapeDtypeStruct((N, 1), jnp.float32),
            jax.ShapeDtypeStruct((N, 1), jnp.float32),
            jax.ShapeDtypeStruct((N, K), jnp.int32),
            jax.ShapeDtypeStruct((1, E), jnp.int32),
            jax.ShapeDtypeStruct((1, E), jnp.int32),
            jax.ShapeDtypeStruct((1, E), jnp.int32),
            jax.ShapeDtypeStruct((1, 64), jnp.int32),
        ],
        scratch_shapes=[pltpu.VMEM((1, E), jnp.int32)],
    )(x, Wg)


def _ffn_body(gid_ref, nblk_ref, x_ref, tok_ref, w1_ref, b1_ref, w2_ref,
              b2_ref, w_ref, ys_ref):
    g = pl.program_id(0)

    @pl.when(g < nblk_ref[0])
    def _():
        onehot = (tok_ref[...] == lax.broadcasted_iota(
            jnp.int32, (BLK, N), 1)).astype(jnp.bfloat16)
        xb = jnp.dot(onehot, x_ref[...], preferred_element_type=jnp.float32
                     ).astype(jnp.bfloat16)
        h = jnp.dot(xb, w1_ref[0].astype(jnp.bfloat16),
                    preferred_element_type=jnp.float32)
        h = jnp.maximum(h + b1_ref[0], 0.0).astype(jnp.bfloat16)
        y = jnp.dot(h, w2_ref[0].astype(jnp.bfloat16),
                    preferred_element_type=jnp.float32)
        ys_ref[...] = (y + b2_ref[0]) * w_ref[...]


def _ffn(x_b, tok_col, W1, b1, W2, b2, w_col, gid, nblk):
    grid_spec = pltpu.PrefetchScalarGridSpec(
        num_scalar_prefetch=2,
        grid=(G,),
        in_specs=[
            pl.BlockSpec((N, D), lambda g, gid, nblk: (0, 0)),
            pl.BlockSpec((BLK, 1), lambda g, gid, nblk: (g, 0)),
            pl.BlockSpec((1, D, F), lambda g, gid, nblk: (gid[g], 0, 0)),
            pl.BlockSpec((1, 1, F), lambda g, gid, nblk: (gid[g], 0, 0)),
            pl.BlockSpec((1, F, D), lambda g, gid, nblk: (gid[g], 0, 0)),
            pl.BlockSpec((1, 1, D), lambda g, gid, nblk: (gid[g], 0, 0)),
            pl.BlockSpec((BLK, 1), lambda g, gid, nblk: (g, 0)),
        ],
        out_specs=pl.BlockSpec((BLK, D), lambda g, gid, nblk: (g, 0)),
    )
    return pl.pallas_call(
        _ffn_body,
        grid_spec=grid_spec,
        out_shape=jax.ShapeDtypeStruct((RPAD, D), jnp.float32),
    )(gid, nblk, x_b, tok_col, W1, b1.reshape(E, 1, F), W2,
      b2.reshape(E, 1, D), w_col)


_NC, _NS = 2, 16
_NW = _NC * _NS

_G_RPW = RPAD // _NW
_G_CH = 40
_G_NCH = _G_RPW // _G_CH

_C_TPW = N // _NW
_C_CH = 32

@functools.cache
def _sc_kernels():
    mesh = plsc.VectorSubcoreMesh(
        core_axis_name="c", subcore_axis_name="s", num_cores=_NC)

    @functools.partial(
        pl.kernel,
        mesh=mesh,
        compiler_params=pltpu.CompilerParams(needs_layout_passes=False),
        out_type=[
            jax.ShapeDtypeStruct((RPAD,), jnp.int32),
            jax.ShapeDtypeStruct((RPAD,), jnp.float32),
        ],
        scratch_types=[
            pltpu.VMEM((N * K,), jnp.int32),
            pltpu.VMEM((N * K,), jnp.float32),
            pltpu.VMEM((_G_RPW,), jnp.int32),
            pltpu.VMEM((_G_RPW,), jnp.float32),
        ],
    )
    def sc_dispatch(dst_hbm, wn_hbm, tok_hbm, wpad_hbm, dst_v, wnv, tok_v, w_v):
        wid = lax.axis_index("s") * _NC + lax.axis_index("c")
        base = wid * _G_RPW
        pltpu.sync_copy(dst_hbm, dst_v)
        pltpu.sync_copy(wn_hbm, wnv)
        zi = jnp.zeros((16,), jnp.int32)
        zf = jnp.zeros((16,), jnp.float32)
        for j in range(_G_RPW // 16):
            tok_v[pl.ds(j * 16, 16)] = zi
            w_v[pl.ds(j * 16, 16)] = zf
        lane = lax.iota(jnp.int32, 16)
        for j in range(N * K // 16):
            dvec = dst_v[pl.ds(j * 16, 16)]
            local = dvec - base
            msk = (local >= 0) & (local < _G_RPW)
            lidx = jnp.clip(local, 0, _G_RPW - 1)
            tvec = lax.shift_right_logical(lane + (j * 16), 1)
            plsc.store_scatter(tok_v, [lidx], tvec, mask=msk)
            plsc.store_scatter(w_v, [lidx], wnv[pl.ds(j * 16, 16)], mask=msk)
        pltpu.sync_copy(tok_v, tok_hbm.at[pl.ds(base, _G_RPW)])
        pltpu.sync_copy(w_v, wpad_hbm.at[pl.ds(base, _G_RPW)])

    @functools.partial(
        pl.kernel,
        mesh=mesh,
        out_type=jax.ShapeDtypeStruct((N, D), jnp.float32),
        scratch_types=[
            pltpu.VMEM((_C_CH,), jnp.int32),
            pltpu.VMEM((_C_CH,), jnp.int32),
            pltpu.VMEM((_C_CH, D), jnp.float32),
            pltpu.VMEM((_C_CH, D), jnp.float32),
            pltpu.SemaphoreType.DMA,
            pltpu.SemaphoreType.DMA,
        ],
    )
    def sc_combine(p0_hbm, p1_hbm, ys_hbm, out_hbm,
                   i0_v, i1_v, a_v, b_v, s0, s1):
        wid = lax.axis_index("s") * _NC + lax.axis_index("c")
        for t in range(_C_TPW // _C_CH):
            base = wid * _C_TPW + t * _C_CH
            pltpu.sync_copy(p0_hbm.at[pl.ds(base, _C_CH)], i0_v)
            pltpu.sync_copy(p1_hbm.at[pl.ds(base, _C_CH)], i1_v)
            cp0 = pltpu.async_copy(ys_hbm.at[i0_v], a_v, s0)
            cp1 = pltpu.async_copy(ys_hbm.at[i1_v], b_v, s1)
            cp0.wait()
            cp1.wait()

            def _row(r, carry):
                def _col(c, cc):
                    sl = pl.ds(c * 16, 16)
                    a_v[r, sl] = a_v[r, sl] + b_v[r, sl]
                    return cc
                return lax.fori_loop(0, D // 16, _col, carry, unroll=4)

            lax.fori_loop(0, _C_CH, _row, 0)
            pltpu.sync_copy(a_v, out_hbm.at[pl.ds(base, _C_CH)])

    return sc_dispatch, sc_combine


def _dispatch(topi, ranks, rowstart, nbk, gid64):
    e_flat = topi.reshape(-1)
    dst = rowstart[0][e_flat] + ranks.reshape(-1)
    gid = gid64[0, :G]
    nblk = nbk[0, E - 1:]
    p = dst.reshape(N, K)
    return dst, gid, nblk, p[:, 0], p[:, 1]


def kernel(x, Wg, W1, b1, W2, b2):
    (probs, topv, topi, wn, ent, mass, ranks, counts,
     rowstart, nbk, gid64) = _router(x, Wg)
    dst, gid, nblk, p0, p1 = _dispatch(topi, ranks, rowstart, nbk, gid64)
    sc_dispatch, sc_combine = _sc_kernels()
    tok, w_pad = sc_dispatch(dst, wn.reshape(-1))
    ys = _ffn(x.astype(jnp.bfloat16), tok.reshape(RPAD, 1), W1, b1, W2, b2,
              w_pad.reshape(RPAD, 1), gid, nblk)
    out = sc_combine(p0, p1, ys)
    return (out, probs, topi, topv, ent.reshape(N), mass.reshape(N))


# speedup vs baseline: 1.5104x; 1.0413x over previous
import functools

import jax
import jax.numpy as jnp
from jax import lax
from jax.experimental import pallas as pl
from jax.experimental.pallas import tpu as pltpu
from jax.experimental.pallas import tpu_sc as plsc

N, D, E, K, F = 2048, 1024, 8, 2, 2048
TEMP = 1.0

BT = 256
BLK = 256
G = 24
RPAD = G * BLK


def _router_body(x_ref, wg_ref, probs_ref, topv_ref, topi_ref, wn_ref,
                 ent_ref, mass_ref, ranks_ref, counts_ref, rowstart_ref,
                 nbk_ref, gid_ref, cnt):
    i = pl.program_id(0)

    @pl.when(i == 0)
    def _():
        cnt[...] = jnp.zeros_like(cnt)

    l = jnp.dot(x_ref[...], wg_ref[...], preferred_element_type=jnp.float32)
    l = l / TEMP
    m = jnp.max(l, axis=1, keepdims=True)
    e = jnp.exp(l - m)
    s = jnp.sum(e, axis=1, keepdims=True)
    p = e / s
    probs_ref[...] = p
    iota = lax.broadcasted_iota(jnp.int32, p.shape, 1)
    v1 = jnp.max(p, axis=1, keepdims=True)
    i1 = jnp.min(jnp.where(p == v1, iota, E), axis=1, keepdims=True)
    p2 = jnp.where(iota == i1, -1.0, p)
    v2 = jnp.max(p2, axis=1, keepdims=True)
    i2 = jnp.min(jnp.where(p2 == v2, iota, E), axis=1, keepdims=True)
    topv_ref[...] = jnp.concatenate([v1, v2], axis=1)
    topi_ref[...] = jnp.concatenate([i1, i2], axis=1)
    ws = v1 + v2
    wn_ref[...] = jnp.concatenate([v1 / ws, v2 / ws], axis=1)
    ent_ref[...] = -jnp.sum(p * jnp.log(p + 1e-9), axis=1, keepdims=True)
    mass_ref[...] = ws
    oh = (i1 == iota).astype(jnp.int32) + (i2 == iota).astype(jnp.int32)
    tri = (lax.broadcasted_iota(jnp.int32, (BT, BT), 0)
           > lax.broadcasted_iota(jnp.int32, (BT, BT), 1)).astype(jnp.float32)
    excl = jnp.dot(tri, oh.astype(jnp.float32),
                   preferred_element_type=jnp.float32,
                   precision=lax.Precision.HIGHEST).astype(jnp.int32) + cnt[...]
    r1 = jnp.sum(jnp.where(i1 == iota, excl, 0), axis=1, keepdims=True)
    r2 = jnp.sum(jnp.where(i2 == iota, excl, 0), axis=1, keepdims=True)
    ranks_ref[...] = jnp.concatenate([r1, r2], axis=1)
    cnt[...] = cnt[...] + jnp.sum(oh, axis=0, keepdims=True)
    counts_ref[...] = cnt[...]

    @pl.when(i == N // BT - 1)
    def _():
        nbf = jnp.floor((cnt[...].astype(jnp.float32) + (BLK - 1)) / BLK)
        ut = (lax.broadcasted_iota(jnp.int32, (E, E), 0)
              <= lax.broadcasted_iota(jnp.int32, (E, E), 1)).astype(jnp.float32)
        cum = jnp.dot(nbf, ut, preferred_element_type=jnp.float32,
                      precision=lax.Precision.HIGHEST)
        rowstart_ref[...] = (BLK * (cum - nbf)).astype(jnp.int32)
        nbk_ref[...] = cum.astype(jnp.int32)
        ident = (lax.broadcasted_iota(jnp.int32, (E, E), 0)
                 == lax.broadcasted_iota(jnp.int32, (E, E), 1)).astype(jnp.float32)
        cum_t = lax.dot_general(ident, cum, (((1,), (1,)), ((), ())),
                                preferred_element_type=jnp.float32,
                                precision=lax.Precision.HIGHEST)
        gg = lax.broadcasted_iota(jnp.int32, (E, 64), 1).astype(jnp.float32)
        cmp = (cum_t <= gg).astype(jnp.float32)
        gidf = jnp.dot(jnp.ones((1, E), jnp.float32), cmp,
                       preferred_element_type=jnp.float32,
                       precision=lax.Precision.HIGHEST)
        gid_ref[...] = jnp.minimum(gidf.astype(jnp.int32), E - 1)


def _router(x, Wg):
    return pl.pallas_call(
        _router_body,
        grid=(N // BT,),
        in_specs=[
            pl.BlockSpec((BT, D), lambda i: (i, 0)),
            pl.BlockSpec((D, E), lambda i: (0, 0)),
        ],
        out_specs=[
            pl.BlockSpec((BT, E), lambda i: (i, 0)),
            pl.BlockSpec((BT, K), lambda i: (i, 0)),
            pl.BlockSpec((BT, K), lambda i: (i, 0)),
            pl.BlockSpec((BT, K), lambda i: (i, 0)),
            pl.BlockSpec((BT, 1), lambda i: (i, 0)),
            pl.BlockSpec((BT, 1), lambda i: (i, 0)),
            pl.BlockSpec((BT, K), lambda i: (i, 0)),
            pl.BlockSpec((1, E), lambda i: (0, 0)),
            pl.BlockSpec((1, E), lambda i: (0, 0)),
            pl.BlockSpec((1, E), lambda i: (0, 0)),
            pl.BlockSpec((1, 64), lambda i: (0, 0)),
        ],
        out_shape=[
            jax.ShapeDtypeStruct((N, E), jnp.float32),
            jax.ShapeDtypeStruct((N, K), jnp.float32),
            jax.ShapeDtypeStruct((N, K), jnp.int32),
            jax.ShapeDtypeStruct((N, K), jnp.float32),
            jax.ShapeDtypeStruct((N, 1), jnp.float32),
            jax.ShapeDtypeStruct((N, 1), jnp.float32),
            jax.ShapeDtypeStruct((N, K), jnp.int32),
            jax.ShapeDtypeStruct((1, E), jnp.int32),
            jax.ShapeDtypeStruct((1, E), jnp.int32),
            jax.ShapeDtypeStruct((1, E), jnp.int32),
            jax.ShapeDtypeStruct((1, 64), jnp.int32),
        ],
        scratch_shapes=[pltpu.VMEM((1, E), jnp.int32)],
    )(x, Wg)


def _ffn_body(gid_ref, nblk_ref, x_ref, tok_ref, w1_ref, b1_ref, w2_ref,
              b2_ref, w_ref, ys_ref):
    g = pl.program_id(0)

    @pl.when(g < nblk_ref[0])
    def _():
        onehot = (tok_ref[...] == lax.broadcasted_iota(
            jnp.int32, (BLK, N), 1)).astype(jnp.bfloat16)
        xb = jnp.dot(onehot, x_ref[...], preferred_element_type=jnp.float32
                     ).astype(jnp.bfloat16)
        h = jnp.dot(xb, w1_ref[0].astype(jnp.bfloat16),
                    preferred_element_type=jnp.float32)
        h = jnp.maximum(h + b1_ref[0], 0.0).astype(jnp.bfloat16)
        y = jnp.dot(h, w2_ref[0].astype(jnp.bfloat16),
                    preferred_element_type=jnp.float32)
        ys_ref[...] = (y + b2_ref[0]) * w_ref[...]


def _ffn(x_b, tok_col, W1, b1, W2, b2, w_col, gid, nblk):
    grid_spec = pltpu.PrefetchScalarGridSpec(
        num_scalar_prefetch=2,
        grid=(G,),
        in_specs=[
            pl.BlockSpec((N, D), lambda g, gid, nblk: (0, 0)),
            pl.BlockSpec((BLK, 1), lambda g, gid, nblk: (g, 0)),
            pl.BlockSpec((1, D, F), lambda g, gid, nblk: (gid[g], 0, 0)),
            pl.BlockSpec((1, 1, F), lambda g, gid, nblk: (gid[g], 0, 0)),
            pl.BlockSpec((1, F, D), lambda g, gid, nblk: (gid[g], 0, 0)),
            pl.BlockSpec((1, 1, D), lambda g, gid, nblk: (gid[g], 0, 0)),
            pl.BlockSpec((BLK, 1), lambda g, gid, nblk: (g, 0)),
        ],
        out_specs=pl.BlockSpec((BLK, D), lambda g, gid, nblk: (g, 0)),
    )
    return pl.pallas_call(
        _ffn_body,
        grid_spec=grid_spec,
        out_shape=jax.ShapeDtypeStruct((RPAD, D), jnp.float32),
    )(gid, nblk, x_b, tok_col, W1, b1.reshape(E, 1, F), W2,
      b2.reshape(E, 1, D), w_col)


_NC, _NS = 2, 16
_NW = _NC * _NS

_G_RPW = RPAD // _NW
_G_CH = 40
_G_NCH = _G_RPW // _G_CH

_C_TPW = N // _NW
_C_CH = 32

@functools.cache
def _sc_kernels():
    mesh = plsc.VectorSubcoreMesh(
        core_axis_name="c", subcore_axis_name="s", num_cores=_NC)

    @functools.partial(
        pl.kernel,
        mesh=mesh,
        compiler_params=pltpu.CompilerParams(needs_layout_passes=False),
        out_type=[
            jax.ShapeDtypeStruct((RPAD,), jnp.int32),
            jax.ShapeDtypeStruct((RPAD,), jnp.float32),
        ],
        scratch_types=[
            pltpu.VMEM((N * K,), jnp.int32),
            pltpu.VMEM((N * K,), jnp.float32),
            pltpu.VMEM((_G_RPW,), jnp.int32),
            pltpu.VMEM((_G_RPW,), jnp.float32),
        ],
    )
    def sc_dispatch(dst_hbm, wn_hbm, tok_hbm, wpad_hbm, dst_v, wnv, tok_v, w_v):
        wid = lax.axis_index("s") * _NC + lax.axis_index("c")
        base = wid * _G_RPW
        pltpu.sync_copy(dst_hbm, dst_v)
        pltpu.sync_copy(wn_hbm, wnv)
        zi = jnp.zeros((16,), jnp.int32)
        zf = jnp.zeros((16,), jnp.float32)
        for j in range(_G_RPW // 16):
            tok_v[pl.ds(j * 16, 16)] = zi
            w_v[pl.ds(j * 16, 16)] = zf
        lane = lax.iota(jnp.int32, 16)
        for j in range(N * K // 16):
            dvec = dst_v[pl.ds(j * 16, 16)]
            local = dvec - base
            msk = (local >= 0) & (local < _G_RPW)
            lidx = jnp.clip(local, 0, _G_RPW - 1)
            tvec = lax.shift_right_logical(lane + (j * 16), 1)
            plsc.store_scatter(tok_v, [lidx], tvec, mask=msk)
            plsc.store_scatter(w_v, [lidx], wnv[pl.ds(j * 16, 16)], mask=msk)
        pltpu.sync_copy(tok_v, tok_hbm.at[pl.ds(base, _G_RPW)])
        pltpu.sync_copy(w_v, wpad_hbm.at[pl.ds(base, _G_RPW)])

    @functools.partial(
        pl.kernel,
        mesh=mesh,
        out_type=jax.ShapeDtypeStruct((N, D), jnp.float32),
        scratch_types=[
            pltpu.VMEM((_C_CH,), jnp.int32),
            pltpu.VMEM((_C_CH,), jnp.int32),
            pltpu.VMEM((_C_CH, D), jnp.float32),
            pltpu.VMEM((_C_CH, D), jnp.float32),
            pltpu.SemaphoreType.DMA,
            pltpu.SemaphoreType.DMA,
        ],
    )
    def sc_combine(p0_hbm, p1_hbm, ys_hbm, out_hbm,
                   i0_v, i1_v, a_v, b_v, s0, s1):
        wid = lax.axis_index("s") * _NC + lax.axis_index("c")
        for t in range(_C_TPW // _C_CH):
            base = wid * _C_TPW + t * _C_CH
            pltpu.sync_copy(p0_hbm.at[pl.ds(base, _C_CH)], i0_v)
            pltpu.sync_copy(p1_hbm.at[pl.ds(base, _C_CH)], i1_v)
            cp0 = pltpu.async_copy(ys_hbm.at[i0_v], a_v, s0)
            cp1 = pltpu.async_copy(ys_hbm.at[i1_v], b_v, s1)
            cp0.wait()
            cp1.wait()

            def _row(r, carry):
                def _col(c, cc):
                    sl = pl.ds(c * 16, 16)
                    a_v[r, sl] = a_v[r, sl] + b_v[r, sl]
                    return cc
                return lax.fori_loop(0, D // 16, _col, carry, unroll=4)

            lax.fori_loop(0, _C_CH, _row, 0)
            pltpu.sync_copy(a_v, out_hbm.at[pl.ds(base, _C_CH)])

    return sc_dispatch, sc_combine


def _dispatch(topi, ranks, rowstart, nbk, gid64):
    e_flat = topi.reshape(-1)
    dst = rowstart[0][e_flat] + ranks.reshape(-1)
    gid = gid64[0, :G]
    nblk = nbk[0, E - 1:]
    p = dst.reshape(N, K)
    return dst, gid, nblk, p[:, 0], p[:, 1]


def kernel(x, Wg, W1, b1, W2, b2):
    (probs, topv, topi, wn, ent, mass, ranks, counts,
     rowstart, nbk, gid64) = _router(x, Wg)
    dst, gid, nblk, p0, p1 = _dispatch(topi, ranks, rowstart, nbk, gid64)
    sc_dispatch, sc_combine = _sc_kernels()
    tok, w_pad = sc_dispatch(dst, wn.reshape(-1))
    ys = _ffn(x.astype(jnp.bfloat16), tok.reshape(RPAD, 1), W1, b1, W2, b2,
              w_pad.reshape(RPAD, 1), gid, nblk)
    out = sc_combine(p0, p1, ys)
    return (out, probs, topi, topv, ent.reshape(N), mass.reshape(N))


# speedup vs baseline: 1.5442x; 1.0224x over previous
import functools

import jax
import jax.numpy as jnp
from jax import lax
from jax.experimental import pallas as pl
from jax.experimental.pallas import tpu as pltpu
from jax.experimental.pallas import tpu_sc as plsc

N, D, E, K, F = 2048, 1024, 8, 2, 2048
TEMP = 1.0

BT = 256
BLK = 256
G = 24
RPAD = G * BLK


def _router_body(x_ref, wg_ref, probs_ref, topv_ref, topi_ref, wn_ref,
                 ent_ref, mass_ref, ranks_ref, counts_ref, rowstart_ref,
                 nbk_ref, gid_ref, cnt):
    i = pl.program_id(0)

    @pl.when(i == 0)
    def _():
        cnt[...] = jnp.zeros_like(cnt)

    l = jnp.dot(x_ref[...], wg_ref[...], preferred_element_type=jnp.float32)
    l = l / TEMP
    m = jnp.max(l, axis=1, keepdims=True)
    e = jnp.exp(l - m)
    s = jnp.sum(e, axis=1, keepdims=True)
    p = e / s
    probs_ref[...] = p
    iota = lax.broadcasted_iota(jnp.int32, p.shape, 1)
    v1 = jnp.max(p, axis=1, keepdims=True)
    i1 = jnp.min(jnp.where(p == v1, iota, E), axis=1, keepdims=True)
    p2 = jnp.where(iota == i1, -1.0, p)
    v2 = jnp.max(p2, axis=1, keepdims=True)
    i2 = jnp.min(jnp.where(p2 == v2, iota, E), axis=1, keepdims=True)
    topv_ref[...] = jnp.concatenate([v1, v2], axis=1)
    topi_ref[...] = jnp.concatenate([i1, i2], axis=1)
    ws = v1 + v2
    wn_ref[...] = jnp.concatenate([v1 / ws, v2 / ws], axis=1)
    ent_ref[...] = -jnp.sum(p * jnp.log(p + 1e-9), axis=1, keepdims=True)
    mass_ref[...] = ws
    oh = (i1 == iota).astype(jnp.int32) + (i2 == iota).astype(jnp.int32)
    tri = (lax.broadcasted_iota(jnp.int32, (BT, BT), 0)
           > lax.broadcasted_iota(jnp.int32, (BT, BT), 1)).astype(jnp.float32)
    excl = jnp.dot(tri, oh.astype(jnp.float32),
                   preferred_element_type=jnp.float32,
                   precision=lax.Precision.HIGHEST).astype(jnp.int32) + cnt[...]
    r1 = jnp.sum(jnp.where(i1 == iota, excl, 0), axis=1, keepdims=True)
    r2 = jnp.sum(jnp.where(i2 == iota, excl, 0), axis=1, keepdims=True)
    ranks_ref[...] = jnp.concatenate([r1, r2], axis=1)
    cnt[...] = cnt[...] + jnp.sum(oh, axis=0, keepdims=True)
    counts_ref[...] = cnt[...]

    @pl.when(i == N // BT - 1)
    def _():
        nbf = jnp.floor((cnt[...].astype(jnp.float32) + (BLK - 1)) / BLK)
        ut = (lax.broadcasted_iota(jnp.int32, (E, E), 0)
              <= lax.broadcasted_iota(jnp.int32, (E, E), 1)).astype(jnp.float32)
        cum = jnp.dot(nbf, ut, preferred_element_type=jnp.float32,
                      precision=lax.Precision.HIGHEST)
        rowstart_ref[...] = (BLK * (cum - nbf)).astype(jnp.int32)
        nbk_ref[...] = cum.astype(jnp.int32)
        ident = (lax.broadcasted_iota(jnp.int32, (E, E), 0)
                 == lax.broadcasted_iota(jnp.int32, (E, E), 1)).astype(jnp.float32)
        cum_t = lax.dot_general(ident, cum, (((1,), (1,)), ((), ())),
                                preferred_element_type=jnp.float32,
                                precision=lax.Precision.HIGHEST)
        gg = lax.broadcasted_iota(jnp.int32, (E, 64), 1).astype(jnp.float32)
        cmp = (cum_t <= gg).astype(jnp.float32)
        gidf = jnp.dot(jnp.ones((1, E), jnp.float32), cmp,
                       preferred_element_type=jnp.float32,
                       precision=lax.Precision.HIGHEST)
        gid_ref[...] = jnp.minimum(gidf.astype(jnp.int32), E - 1)


def _router(x, Wg):
    return pl.pallas_call(
        _router_body,
        grid=(N // BT,),
        in_specs=[
            pl.BlockSpec((BT, D), lambda i: (i, 0)),
            pl.BlockSpec((D, E), lambda i: (0, 0)),
        ],
        out_specs=[
            pl.BlockSpec((BT, E), lambda i: (i, 0)),
            pl.BlockSpec((BT, K), lambda i: (i, 0)),
            pl.BlockSpec((BT, K), lambda i: (i, 0)),
            pl.BlockSpec((BT, K), lambda i: (i, 0)),
            pl.BlockSpec((BT, 1), lambda i: (i, 0)),
            pl.BlockSpec((BT, 1), lambda i: (i, 0)),
            pl.BlockSpec((BT, K), lambda i: (i, 0)),
            pl.BlockSpec((1, E), lambda i: (0, 0)),
            pl.BlockSpec((1, E), lambda i: (0, 0)),
            pl.BlockSpec((1, E), lambda i: (0, 0)),
            pl.BlockSpec((1, 64), lambda i: (0, 0)),
        ],
        out_shape=[
            jax.ShapeDtypeStruct((N, E), jnp.float32),
            jax.ShapeDtypeStruct((N, K), jnp.float32),
            jax.ShapeDtypeStruct((N, K), jnp.int32),
            jax.ShapeDtypeStruct((N, K), jnp.float32),
            jax.ShapeDtypeStruct((N, 1), jnp.float32),
            jax.ShapeDtypeStruct((N, 1), jnp.float32),
            jax.ShapeDtypeStruct((N, K), jnp.int32),
            jax.ShapeDtypeStruct((1, E), jnp.int32),
            jax.ShapeDtypeStruct((1, E), jnp.int32),
            jax.ShapeDtypeStruct((1, E), jnp.int32),
            jax.ShapeDtypeStruct((1, 64), jnp.int32),
        ],
        scratch_shapes=[pltpu.VMEM((1, E), jnp.int32)],
    )(x, Wg)


def _ffn_body(gid_ref, nblk_ref, x_ref, tok_ref, w1_ref, b1_ref, w2_ref,
              b2_ref, w_ref, ys_ref):
    g = pl.program_id(0)

    @pl.when(g < nblk_ref[0])
    def _():
        onehot = (tok_ref[...] == lax.broadcasted_iota(
            jnp.int32, (BLK, N), 1)).astype(jnp.bfloat16)
        xb = jnp.dot(onehot, x_ref[...], preferred_element_type=jnp.float32
                     ).astype(jnp.bfloat16)
        h = jnp.dot(xb, w1_ref[0].astype(jnp.bfloat16),
                    preferred_element_type=jnp.float32)
        h = jnp.maximum(h + b1_ref[0], 0.0).astype(jnp.bfloat16)
        y = jnp.dot(h, w2_ref[0].astype(jnp.bfloat16),
                    preferred_element_type=jnp.float32)
        ys_ref[...] = (y + b2_ref[0]) * w_ref[...]


def _ffn(x_b, tok_col, W1, b1, W2, b2, w_col, gid, nblk):
    grid_spec = pltpu.PrefetchScalarGridSpec(
        num_scalar_prefetch=2,
        grid=(G,),
        in_specs=[
            pl.BlockSpec((N, D), lambda g, gid, nblk: (0, 0)),
            pl.BlockSpec((BLK, 1), lambda g, gid, nblk: (g, 0)),
            pl.BlockSpec((1, D, F), lambda g, gid, nblk: (gid[g], 0, 0)),
            pl.BlockSpec((1, 1, F), lambda g, gid, nblk: (gid[g], 0, 0)),
            pl.BlockSpec((1, F, D), lambda g, gid, nblk: (gid[g], 0, 0)),
            pl.BlockSpec((1, 1, D), lambda g, gid, nblk: (gid[g], 0, 0)),
            pl.BlockSpec((BLK, 1), lambda g, gid, nblk: (g, 0)),
        ],
        out_specs=pl.BlockSpec((BLK, D), lambda g, gid, nblk: (g, 0)),
    )
    return pl.pallas_call(
        _ffn_body,
        grid_spec=grid_spec,
        out_shape=jax.ShapeDtypeStruct((RPAD, D), jnp.float32),
    )(gid, nblk, x_b, tok_col, W1, b1.reshape(E, 1, F), W2,
      b2.reshape(E, 1, D), w_col)


_NC, _NS = 2, 16
_NW = _NC * _NS

_G_RPW = RPAD // _NW
_G_CH = 40
_G_NCH = _G_RPW // _G_CH

_C_TPW = N // _NW
_C_CH = 16
_C_NCH = _C_TPW // _C_CH

@functools.cache
def _sc_kernels():
    mesh = plsc.VectorSubcoreMesh(
        core_axis_name="c", subcore_axis_name="s", num_cores=_NC)

    @functools.partial(
        pl.kernel,
        mesh=mesh,
        compiler_params=pltpu.CompilerParams(needs_layout_passes=False),
        out_type=[
            jax.ShapeDtypeStruct((RPAD,), jnp.int32),
            jax.ShapeDtypeStruct((RPAD,), jnp.float32),
        ],
        scratch_types=[
            pltpu.VMEM((N * K,), jnp.int32),
            pltpu.VMEM((N * K,), jnp.float32),
            pltpu.VMEM((_G_RPW,), jnp.int32),
            pltpu.VMEM((_G_RPW,), jnp.float32),
        ],
    )
    def sc_dispatch(dst_hbm, wn_hbm, tok_hbm, wpad_hbm, dst_v, wnv, tok_v, w_v):
        wid = lax.axis_index("s") * _NC + lax.axis_index("c")
        base = wid * _G_RPW
        pltpu.sync_copy(dst_hbm, dst_v)
        pltpu.sync_copy(wn_hbm, wnv)
        zi = jnp.zeros((16,), jnp.int32)
        zf = jnp.zeros((16,), jnp.float32)
        for j in range(_G_RPW // 16):
            tok_v[pl.ds(j * 16, 16)] = zi
            w_v[pl.ds(j * 16, 16)] = zf
        lane = lax.iota(jnp.int32, 16)
        for j in range(N * K // 16):
            dvec = dst_v[pl.ds(j * 16, 16)]
            local = dvec - base
            msk = (local >= 0) & (local < _G_RPW)
            lidx = jnp.clip(local, 0, _G_RPW - 1)
            tvec = lax.shift_right_logical(lane + (j * 16), 1)
            plsc.store_scatter(tok_v, [lidx], tvec, mask=msk)
            plsc.store_scatter(w_v, [lidx], wnv[pl.ds(j * 16, 16)], mask=msk)
        pltpu.sync_copy(tok_v, tok_hbm.at[pl.ds(base, _G_RPW)])
        pltpu.sync_copy(w_v, wpad_hbm.at[pl.ds(base, _G_RPW)])

    @functools.partial(
        pl.kernel,
        mesh=mesh,
        out_type=jax.ShapeDtypeStruct((N, D), jnp.float32),
        scratch_types=[
            pltpu.VMEM((_C_TPW,), jnp.int32),
            pltpu.VMEM((_C_TPW,), jnp.int32),
            pltpu.VMEM((_C_CH, D), jnp.float32),
            pltpu.VMEM((_C_CH, D), jnp.float32),
            pltpu.VMEM((_C_CH, D), jnp.float32),
            pltpu.VMEM((_C_CH, D), jnp.float32),
            pltpu.VMEM((_C_CH, D), jnp.float32),
            pltpu.VMEM((_C_CH, D), jnp.float32),
            pltpu.SemaphoreType.DMA,
            pltpu.SemaphoreType.DMA,
            pltpu.SemaphoreType.DMA,
            pltpu.SemaphoreType.DMA,
            pltpu.SemaphoreType.DMA,
            pltpu.SemaphoreType.DMA,
            pltpu.SemaphoreType.DMA,
            pltpu.SemaphoreType.DMA,
            pltpu.SemaphoreType.DMA,
        ],
    )
    def sc_combine(p0_hbm, p1_hbm, ys_hbm, out_hbm, i0_v, i1_v,
                   a0, a1, a2, b0, b1, b2,
                   sa0, sa1, sa2, sb0, sb1, sb2, sw0, sw1, sw2):
        wid = lax.axis_index("s") * _NC + lax.axis_index("c")
        base = wid * _C_TPW
        pltpu.sync_copy(p0_hbm.at[pl.ds(base, _C_TPW)], i0_v)
        pltpu.sync_copy(p1_hbm.at[pl.ds(base, _C_TPW)], i1_v)
        av, bv = [a0, a1, a2], [b0, b1, b2]
        sa, sb, sw = [sa0, sa1, sa2], [sb0, sb1, sb2], [sw0, sw1, sw2]
        ga = [None] * _C_NCH
        gb = [None] * _C_NCH
        wcp = [None] * _C_NCH

        def _issue(t):
            s = t % 3
            sl = pl.ds(t * _C_CH, _C_CH)
            ga[t] = pltpu.async_copy(ys_hbm.at[i0_v.at[sl]], av[s], sa[s])
            gb[t] = pltpu.async_copy(ys_hbm.at[i1_v.at[sl]], bv[s], sb[s])

        _issue(0)
        _issue(1)
        for t in range(_C_NCH):
            s = t % 3
            ga[t].wait()
            gb[t].wait()
            aa, bb = av[s], bv[s]

            def _row(r, carry):
                def _col(c, cc):
                    sl = pl.ds(c * 16, 16)
                    aa[r, sl] = aa[r, sl] + bb[r, sl]
                    return cc
                return lax.fori_loop(0, D // 16, _col, carry, unroll=4)

            lax.fori_loop(0, _C_CH, _row, 0)
            wcp[t] = pltpu.async_copy(
                aa, out_hbm.at[pl.ds(base + t * _C_CH, _C_CH)], sw[s])
            if t + 2 < _C_NCH:
                if t >= 1:
                    wcp[t - 1].wait()
                _issue(t + 2)
        wcp[_C_NCH - 3].wait()
        wcp[_C_NCH - 2].wait()
        wcp[_C_NCH - 1].wait()

    return sc_dispatch, sc_combine


def _dispatch(topi, ranks, rowstart, nbk, gid64):
    e_flat = topi.reshape(-1)
    dst = rowstart[0][e_flat] + ranks.reshape(-1)
    gid = gid64[0, :G]
    nblk = nbk[0, E - 1:]
    p = dst.reshape(N, K)
    return dst, gid, nblk, p[:, 0], p[:, 1]


def kernel(x, Wg, W1, b1, W2, b2):
    (probs, topv, topi, wn, ent, mass, ranks, counts,
     rowstart, nbk, gid64) = _router(x, Wg)
    dst, gid, nblk, p0, p1 = _dispatch(topi, ranks, rowstart, nbk, gid64)
    sc_dispatch, sc_combine = _sc_kernels()
    tok, w_pad = sc_dispatch(dst, wn.reshape(-1))
    ys = _ffn(x.astype(jnp.bfloat16), tok.reshape(RPAD, 1), W1, b1, W2, b2,
              w_pad.reshape(RPAD, 1), gid, nblk)
    out = sc_combine(p0, p1, ys)
    return (out, probs, topi, topv, ent.reshape(N), mass.reshape(N))
